# Initial kernel scaffold; baseline (speedup 1.0000x reference)
import jax
import jax.numpy as jnp
from jax.experimental import pallas as pl


def kernel(x, W, b):
    raise NotImplementedError("write your pallas kernel here")


# speedup vs baseline: 2.2458x; 2.2458x over previous
import functools

import jax
import jax.numpy as jnp
from jax import lax
from jax.experimental import pallas as pl
from jax.experimental.pallas import tpu as pltpu
from jax.experimental.pallas import tpu_sc as plsc

L = 16
NC, NS = 2, 16
NW = NC * NS
HW = 256
B_DIM, C_DIM = 64, 768
ROWS = B_DIM * C_DIM
ROWS_PER_TILE = ROWS // NW
CHUNK = 128
NCHUNK = ROWS_PER_TILE // CHUNK
NCLS = 21841


def _vs(v, desc):
    return plsc.sort_key_val(v, v, descending=desc)[0]


def _bmerge(vs, desc):
    m = len(vs)
    if m == 1:
        return [_vs(vs[0], desc)]
    half = m // 2
    lo, hi = [], []
    for i in range(half):
        a, b = vs[i], vs[i + half]
        mn = jnp.minimum(a, b)
        mx = jnp.maximum(a, b)
        lo.append(mx if desc else mn)
        hi.append(mn if desc else mx)
    return _bmerge(lo, desc) + _bmerge(hi, desc)


def _bsort(vs, desc):
    if len(vs) == 1:
        return [_vs(vs[0], desc)]
    half = len(vs) // 2
    return _bmerge(_bsort(vs[:half], False) + _bsort(vs[half:], True), desc)


def _row_median(vs):
    a = _bsort(vs[:8], False)
    b = _bsort(vs[8:], True)
    u = jnp.maximum(a[0], b[0])
    for i in range(1, 8):
        u = jnp.minimum(u, jnp.maximum(a[i], b[i]))
    return _vs(u, False)


def _median_sc(xflat):
    mesh = plsc.VectorSubcoreMesh(
        core_axis_name="c", subcore_axis_name="s",
        num_cores=NC, num_subcores=NS)

    @functools.partial(
        pl.kernel,
        out_type=jax.ShapeDtypeStruct((ROWS,), jnp.float32),
        mesh=mesh,
        scratch_types=[
            pltpu.VMEM((CHUNK * HW,), jnp.float32),
            pltpu.VMEM((ROWS_PER_TILE,), jnp.float32),
            pltpu.SemaphoreType.DMA,
        ],
        compiler_params=pltpu.CompilerParams(needs_layout_passes=False),
    )
    def med_kernel(x_hbm, out_hbm, buf, med, sem):
        wid = lax.axis_index("s") * NC + lax.axis_index("c")
        tile_base = wid * ROWS_PER_TILE
        mask0 = lax.iota(jnp.int32, L) == 0

        def chunk_body(c, carry):
            pltpu.async_copy(
                x_hbm.at[pl.ds((tile_base + c * CHUNK) * HW, CHUNK * HW)],
                buf, sem).wait()

            def row_body(r, rcarry):
                base = r * HW
                vs = [buf[pl.ds(base + j * L, L)] for j in range(16)]
                u = _row_median(vs)
                idx = jnp.broadcast_to(c * CHUNK + r, (L,)).astype(jnp.int32)
                plsc.store_scatter(med, [idx], u, mask=mask0)
                return rcarry

            lax.fori_loop(0, CHUNK, row_body, 0)
            return carry

        lax.fori_loop(0, NCHUNK, chunk_body, 0)
        pltpu.sync_copy(med, out_hbm.at[pl.ds(tile_base, ROWS_PER_TILE)])

    return med_kernel(xflat)


def _linear_tc(med2, W, b2):
    BN = 2048

    def mm_kernel(med_ref, w_ref, b_ref, o_ref):
        o_ref[...] = lax.dot_general(
            med_ref[...], w_ref[...], (((1,), (1,)), ((), ())),
            preferred_element_type=jnp.float32) + b_ref[...]

    return pl.pallas_call(
        mm_kernel,
        grid=(pl.cdiv(NCLS, BN),),
        in_specs=[
            pl.BlockSpec((B_DIM, C_DIM), lambda i: (0, 0)),
            pl.BlockSpec((BN, C_DIM), lambda i: (i, 0)),
            pl.BlockSpec((1, BN), lambda i: (0, i)),
        ],
        out_specs=pl.BlockSpec((B_DIM, BN), lambda i: (0, i)),
        out_shape=jax.ShapeDtypeStruct((B_DIM, NCLS), jnp.float32),
    )(med2, W, b2)


def kernel(x, W, b):
    xflat = x.reshape(ROWS * HW)
    med = _median_sc(xflat)
    return _linear_tc(med.reshape(B_DIM, C_DIM), W, b.reshape(1, NCLS))


# speedup vs baseline: 3.1143x; 1.3867x over previous
import functools

import jax
import jax.numpy as jnp
from jax import lax
from jax.experimental import pallas as pl
from jax.experimental.pallas import tpu as pltpu
from jax.experimental.pallas import tpu_sc as plsc

L = 16
NC, NS = 2, 16
NW = NC * NS
HW = 256
B_DIM, C_DIM = 64, 768
CT = C_DIM // 128
UNITS = B_DIM * CT
UNITS_PER_TILE = UNITS // NW
NCLS = 21841


def _vs(v, desc):
    return plsc.sort_key_val(v, v, descending=desc)[0]


def _bmerge(vs, desc):
    m = len(vs)
    if m == 1:
        return [_vs(vs[0], desc)]
    half = m // 2
    lo, hi = [], []
    for i in range(half):
        a, b = vs[i], vs[i + half]
        mn = jnp.minimum(a, b)
        mx = jnp.maximum(a, b)
        lo.append(mx if desc else mn)
        hi.append(mn if desc else mx)
    return _bmerge(lo, desc) + _bmerge(hi, desc)


def _bsort(vs, desc):
    if len(vs) == 1:
        return [_vs(vs[0], desc)]
    half = len(vs) // 2
    return _bmerge(_bsort(vs[:half], False) + _bsort(vs[half:], True), desc)


def _row_median(vs):
    a = _bsort(vs[:8], False)
    b = _bsort(vs[8:], True)
    u = jnp.maximum(a[0], b[0])
    for i in range(1, 8):
        u = jnp.minimum(u, jnp.maximum(a[i], b[i]))
    return _vs(u, False)


def _median_sc(xt):
    mesh = plsc.VectorSubcoreMesh(
        core_axis_name="c", subcore_axis_name="s",
        num_cores=NC, num_subcores=NS)

    @functools.partial(
        pl.kernel,
        out_type=jax.ShapeDtypeStruct((B_DIM, C_DIM), jnp.float32),
        mesh=mesh,
        scratch_types=[
            pltpu.VMEM((HW, 128), jnp.float32),
            pltpu.VMEM((128,), jnp.float32),
            pltpu.SemaphoreType.DMA,
        ],
        compiler_params=pltpu.CompilerParams(
            needs_layout_passes=False, use_tc_tiling_on_sc=True),
    )
    def med_kernel(x_hbm, out_hbm, buf, medv, sem):
        wid = lax.axis_index("s") * NC + lax.axis_index("c")
        iota = lax.iota(jnp.int32, L)
        mask0 = iota == 0

        def unit_body(k, carry):
            u = wid * UNITS_PER_TILE + k
            b = u // CT
            ct = u % CT
            pltpu.async_copy(
                x_hbm.at[b, :, pl.ds(ct * 128, 128)], buf, sem).wait()

            def ch_body(c, cc):
                cidx = jnp.broadcast_to(c, (L,)).astype(jnp.int32)
                vs = [plsc.load_gather(buf, [iota + (16 * j), cidx])
                      for j in range(16)]
                med16 = _row_median(vs)
                plsc.store_scatter(medv, [cidx], med16, mask=mask0)
                return cc

            lax.fori_loop(0, 128, ch_body, 0)
            pltpu.sync_copy(medv, out_hbm.at[b, pl.ds(ct * 128, 128)])
            return carry

        lax.fori_loop(0, UNITS_PER_TILE, unit_body, 0)

    return med_kernel(xt)


def _linear_tc(med2, W, b2):
    BN = 2048

    def mm_kernel(med_ref, w_ref, b_ref, o_ref):
        o_ref[...] = lax.dot_general(
            med_ref[...], w_ref[...], (((1,), (1,)), ((), ())),
            preferred_element_type=jnp.float32) + b_ref[...]

    return pl.pallas_call(
        mm_kernel,
        grid=(pl.cdiv(NCLS, BN),),
        in_specs=[
            pl.BlockSpec((B_DIM, C_DIM), lambda i: (0, 0)),
            pl.BlockSpec((BN, C_DIM), lambda i: (i, 0)),
            pl.BlockSpec((1, BN), lambda i: (0, i)),
        ],
        out_specs=pl.BlockSpec((B_DIM, BN), lambda i: (0, i)),
        out_shape=jax.ShapeDtypeStruct((B_DIM, NCLS), jnp.float32),
    )(med2, W, b2)


def kernel(x, W, b):
    xt = x.transpose(0, 2, 3, 1).reshape(B_DIM, HW, C_DIM)
    med = _median_sc(xt)
    return _linear_tc(med, W, b.reshape(1, NCLS))
